# Initial kernel scaffold; baseline (speedup 1.0000x reference)
import functools

import jax
import jax.numpy as jnp
from jax import lax
from jax.experimental import pallas as pl
from jax.experimental.pallas import tpu as pltpu
from jax.experimental.pallas import tpu_sc as plsc

N = 10000
E = 320000
D = 128
COLS = D // 2
NS = 16
ROWS_PER_TILE = N // NS
EDGES_PER_TILE = E // NS
CHUNK = 125
NCHUNK = EDGES_PER_TILE // CHUNK


def _sc_aggregate(x, src3, dst3):
  mesh = plsc.VectorSubcoreMesh(core_axis_name="c", subcore_axis_name="s")

  @functools.partial(
      pl.kernel,
      mesh=mesh,
      out_type=jax.ShapeDtypeStruct((N, D), jnp.float32),
      scratch_types=[
          pltpu.VMEM_SHARED((N, COLS), jnp.float32),
          pltpu.VMEM_SHARED((N, COLS), jnp.float32),
          pltpu.VMEM((NCHUNK, CHUNK), jnp.int32),
          pltpu.VMEM((NCHUNK, CHUNK), jnp.int32),
          pltpu.VMEM((CHUNK, COLS), jnp.float32),
          pltpu.SemaphoreType.DMA,
      ],
  )
  def k(x_hbm, src_hbm, dst_hbm, h_hbm, x_s, agg_s, src_v, dst_v, rows_v, sem):
    c = lax.axis_index("c")
    s = lax.axis_index("s")
    r0 = s * ROWS_PER_TILE
    c0 = c * COLS
    pltpu.sync_copy(x_hbm.at[pl.ds(r0, ROWS_PER_TILE), pl.ds(c0, COLS)],
                    x_s.at[pl.ds(r0, ROWS_PER_TILE)])
    pltpu.sync_copy(x_hbm.at[pl.ds(r0, ROWS_PER_TILE), pl.ds(c0, COLS)],
                    agg_s.at[pl.ds(r0, ROWS_PER_TILE)])
    pltpu.sync_copy(src_hbm.at[s], src_v)
    pltpu.sync_copy(dst_hbm.at[s], dst_v)
    plsc.subcore_barrier()

    def step(j, carry):
      pltpu.async_copy(x_s.at[src_v.at[j]], rows_v, sem).wait()
      pltpu.sync_copy(rows_v, agg_s.at[dst_v.at[j]], add=True)
      return carry

    lax.fori_loop(0, NCHUNK, step, 0)
    plsc.subcore_barrier()
    pltpu.sync_copy(agg_s.at[pl.ds(r0, ROWS_PER_TILE)],
                    h_hbm.at[pl.ds(r0, ROWS_PER_TILE), pl.ds(c0, COLS)])

  return k(x, src3, dst3)


def _mlp_body(h_ref, w1_ref, b1_ref, w2_ref, b2_ref, o_ref):
  h = h_ref[...]
  a = jnp.dot(h, w1_ref[...], preferred_element_type=jnp.float32) + b1_ref[...]
  a = jnp.maximum(a, 0.0)
  o_ref[...] = jnp.dot(a, w2_ref[...], preferred_element_type=jnp.float32) + b2_ref[...]


def _mlp(h, W1, b1, W2, b2):
  blk = 1000
  return pl.pallas_call(
      _mlp_body,
      grid=(N // blk,),
      in_specs=[
          pl.BlockSpec((blk, D), lambda i: (i, 0)),
          pl.BlockSpec((D, D), lambda i: (0, 0)),
          pl.BlockSpec((1, D), lambda i: (0, 0)),
          pl.BlockSpec((D, D), lambda i: (0, 0)),
          pl.BlockSpec((1, D), lambda i: (0, 0)),
      ],
      out_specs=pl.BlockSpec((blk, D), lambda i: (i, 0)),
      out_shape=jax.ShapeDtypeStruct((N, D), jnp.float32),
  )(h, W1, b1, W2, b2)


def kernel(x, edge_index, W1, b1, W2, b2):
  src3 = edge_index[0].reshape(NS, NCHUNK, CHUNK)
  dst3 = edge_index[1].reshape(NS, NCHUNK, CHUNK)
  h = _sc_aggregate(x, src3, dst3)
  return _mlp(h, W1, b1.reshape(1, D), W2, b2.reshape(1, D))


# speedup vs baseline: 7.6460x; 7.6460x over previous
import functools

import jax
import jax.numpy as jnp
from jax import lax
from jax.experimental import pallas as pl
from jax.experimental.pallas import tpu as pltpu
from jax.experimental.pallas import tpu_sc as plsc

N = 10000
E = 320000
D = 128
COLS = D // 2
NS = 16
ROWS_PER_TILE = N // NS
EDGES_PER_TILE = E // NS
CHUNK = 125
NCHUNK = EDGES_PER_TILE // CHUNK


def _sc_aggregate(x, src3, dst3):
  mesh = plsc.VectorSubcoreMesh(core_axis_name="c", subcore_axis_name="s")

  @functools.partial(
      pl.kernel,
      mesh=mesh,
      compiler_params=pltpu.CompilerParams(use_tc_tiling_on_sc=False),
      out_type=jax.ShapeDtypeStruct((N, D), jnp.float32),
      scratch_types=[
          pltpu.VMEM_SHARED((N, COLS), jnp.float32),
          pltpu.VMEM_SHARED((N, COLS), jnp.float32),
          pltpu.VMEM((NCHUNK, CHUNK), jnp.int32),
          pltpu.VMEM((NCHUNK, CHUNK), jnp.int32),
          pltpu.VMEM((CHUNK, COLS), jnp.float32),
          pltpu.SemaphoreType.DMA,
      ],
  )
  def k(x_hbm, src_hbm, dst_hbm, h_hbm, x_s, agg_s, src_v, dst_v, rows_v, sem):
    c = lax.axis_index("c")
    s = lax.axis_index("s")
    r0 = s * ROWS_PER_TILE
    c0 = c * COLS
    pltpu.sync_copy(x_hbm.at[pl.ds(r0, ROWS_PER_TILE), pl.ds(c0, COLS)],
                    x_s.at[pl.ds(r0, ROWS_PER_TILE)])
    pltpu.sync_copy(x_hbm.at[pl.ds(r0, ROWS_PER_TILE), pl.ds(c0, COLS)],
                    agg_s.at[pl.ds(r0, ROWS_PER_TILE)])
    pltpu.sync_copy(src_hbm.at[s], src_v)
    pltpu.sync_copy(dst_hbm.at[s], dst_v)
    plsc.subcore_barrier()

    def step(j, carry):
      pltpu.async_copy(x_s.at[src_v.at[j]], rows_v, sem).wait()
      pltpu.sync_copy(rows_v, agg_s.at[dst_v.at[j]], add=True)
      return carry

    lax.fori_loop(0, NCHUNK, step, 0)
    plsc.subcore_barrier()
    pltpu.sync_copy(agg_s.at[pl.ds(r0, ROWS_PER_TILE)],
                    h_hbm.at[pl.ds(r0, ROWS_PER_TILE), pl.ds(c0, COLS)])

  return k(x, src3, dst3)


def _mlp_body(h_ref, w1_ref, b1_ref, w2_ref, b2_ref, o_ref):
  h = h_ref[...]
  a = jnp.dot(h, w1_ref[...], preferred_element_type=jnp.float32) + b1_ref[...]
  a = jnp.maximum(a, 0.0)
  o_ref[...] = jnp.dot(a, w2_ref[...], preferred_element_type=jnp.float32) + b2_ref[...]


def _mlp(h, W1, b1, W2, b2):
  blk = 1000
  return pl.pallas_call(
      _mlp_body,
      grid=(N // blk,),
      in_specs=[
          pl.BlockSpec((blk, D), lambda i: (i, 0)),
          pl.BlockSpec((D, D), lambda i: (0, 0)),
          pl.BlockSpec((1, D), lambda i: (0, 0)),
          pl.BlockSpec((D, D), lambda i: (0, 0)),
          pl.BlockSpec((1, D), lambda i: (0, 0)),
      ],
      out_specs=pl.BlockSpec((blk, D), lambda i: (i, 0)),
      out_shape=jax.ShapeDtypeStruct((N, D), jnp.float32),
  )(h, W1, b1, W2, b2)


def kernel(x, edge_index, W1, b1, W2, b2):
  src3 = edge_index[0].reshape(NS, NCHUNK, CHUNK)
  dst3 = edge_index[1].reshape(NS, NCHUNK, CHUNK)
  h = _sc_aggregate(x, src3, dst3)
  return _mlp(h, W1, b1.reshape(1, D), W2, b2.reshape(1, D))
